# baseline (speedup 1.0000x reference)
import jax
import jax.numpy as jnp
from jax import lax
from jax.experimental import pallas as pl
from jax.experimental.pallas import tpu as pltpu
from jax.experimental.pallas import tpu_sc as plsc

NUM_ENT = 1000000
NUM_REL = 1000
D = 64
B = 16384

_info = plsc.get_sparse_core_info()
NC, NS, L = _info.num_cores, _info.num_subcores, _info.num_lanes
NW = NC * NS
BW = B // NW
CH = 128
NCH = BW // CH


def _body(h_hbm, r_hbm, t_hbm, e_hbm, rtab_hbm, out_hbm,
          h_v, r_v, t_v, eh_v, rr_v, et_v, out_v, sem):
    wid = lax.axis_index("s") * NC + lax.axis_index("c")
    base = wid * BW

    pltpu.sync_copy(h_hbm.at[wid], h_v)
    pltpu.sync_copy(r_hbm.at[wid], r_v)
    pltpu.sync_copy(t_hbm.at[wid], t_v)

    copies = []
    for j in range(NCH):
        rows = pl.ds(j * CH, CH)
        copies.append(pltpu.async_copy(e_hbm.at[h_v.at[j]], eh_v.at[rows], sem))
        copies.append(pltpu.async_copy(rtab_hbm.at[r_v.at[j]], rr_v.at[rows], sem))
        copies.append(pltpu.async_copy(e_hbm.at[t_v.at[j]], et_v.at[rows], sem))
    for c in copies:
        c.wait()

    lanes = lax.iota(jnp.int32, L)

    def group(g, _):
        ridx = g * L + lanes
        def dstep(d, acc):
            dsplat = jnp.zeros((L,), jnp.int32) + d
            gh = plsc.load_gather(eh_v, [ridx, dsplat])
            gr = plsc.load_gather(rr_v, [ridx, dsplat])
            gt = plsc.load_gather(et_v, [ridx, dsplat])
            return acc + jnp.abs(gh + gr - gt)
        acc = lax.fori_loop(0, D, dstep, jnp.zeros((L,), jnp.float32))
        out_v[pl.ds(g * L, L)] = acc
        return 0

    lax.fori_loop(0, BW // L, group, 0)

    pltpu.sync_copy(out_v, out_hbm.at[pl.ds(base, BW)])


def kernel(h, r, t, E, R):
    h = h.astype(jnp.int32).reshape(NW, NCH, CH)
    r = r.astype(jnp.int32).reshape(NW, NCH, CH)
    t = t.astype(jnp.int32).reshape(NW, NCH, CH)

    mesh = plsc.VectorSubcoreMesh(core_axis_name="c", subcore_axis_name="s")
    run = pl.kernel(
        _body,
        out_type=jax.ShapeDtypeStruct((B,), jnp.float32),
        mesh=mesh,
        compiler_params=pltpu.CompilerParams(
            needs_layout_passes=False, use_tc_tiling_on_sc=False),
        scratch_types=[
            pltpu.VMEM((NCH, CH), jnp.int32),
            pltpu.VMEM((NCH, CH), jnp.int32),
            pltpu.VMEM((NCH, CH), jnp.int32),
            pltpu.VMEM((BW, D), jnp.float32),
            pltpu.VMEM((BW, D), jnp.float32),
            pltpu.VMEM((BW, D), jnp.float32),
            pltpu.VMEM((BW,), jnp.float32),
            pltpu.SemaphoreType.DMA,
        ],
    )
    return run(h, r, t, E, R)
